# Initial kernel scaffold; baseline (speedup 1.0000x reference)
import jax
import jax.numpy as jnp
from jax.experimental import pallas as pl


def kernel(pos, x, batch, W1, b1, W2, b2, W3, b3, W4, b4, W5, b5, W6, b6, W7, b7, W8, b8, W9, b9):
    raise NotImplementedError("write your pallas kernel here")


# speedup vs baseline: 10.5880x; 10.5880x over previous
import functools

import jax
import jax.numpy as jnp
from jax import lax
from jax.experimental import pallas as pl
from jax.experimental.pallas import tpu as pltpu
from jax.experimental.pallas import tpu_sc as plsc

N = 8192
K = 20
NUM_GRAPHS = 8
R = 512
HUGE = 3.0e38


CW = 512
IMAX = 2**31 - 1


def _knn_body(clo_ref, chi_ref, feat_ref, featT_ref, bcol_ref, brow_ref,
              idx_ref, dist_ref):
    pid = pl.program_id(0)
    c0 = clo_ref[pid]
    c1 = chi_ref[pid]
    f_r = feat_ref[...]
    sq_r = jnp.sum(f_r * f_r, axis=1, keepdims=True)
    bcol = bcol_ref[...]

    def compute_chunk(c, carry):
        sl = pl.ds(c * CW, CW)
        f_t = featT_ref[:, sl]
        d = lax.dot_general(f_r, f_t, (((1,), (0,)), ((), ())),
                            preferred_element_type=jnp.float32)
        sq_c = jnp.sum(f_t * f_t, axis=0, keepdims=True)
        dist = sq_r - 2.0 * d + sq_c
        dist = jnp.where(bcol != brow_ref[:, sl], 1e10, dist)
        dist_ref[:, sl] = dist
        return carry

    lax.fori_loop(c0, c1, compute_chunk, 0)

    kio = lax.broadcasted_iota(jnp.int32, (R, 32), 1)

    def select_one(t, sel):
        def scan_chunk(c, carry):
            m, amin = carry
            sl = pl.ds(c * CW, CW)
            dch = dist_ref[:, sl]
            colio = lax.broadcasted_iota(jnp.int32, (R, CW), 1) + c * CW
            mch = jnp.min(dch, axis=1, keepdims=True)
            ach = jnp.min(jnp.where(dch == mch, colio, IMAX),
                          axis=1, keepdims=True)
            amin = jnp.where(mch < m, ach, amin)
            return jnp.minimum(m, mch), amin

        m, amin = lax.fori_loop(
            c0, c1, scan_chunk,
            (jnp.full((R, 1), HUGE, jnp.float32), jnp.zeros((R, 1), jnp.int32)))

        def remove_chunk(c, carry):
            sl = pl.ds(c * CW, CW)
            colio = lax.broadcasted_iota(jnp.int32, (R, CW), 1) + c * CW
            dist_ref[:, sl] = jnp.where(colio == amin, HUGE, dist_ref[:, sl])
            return carry

        lax.fori_loop(c0, c1, remove_chunk, 0)
        return jnp.where(kio == t, amin, sel)

    sel = lax.fori_loop(0, K, select_one, jnp.zeros((R, 32), jnp.int32))
    idx_ref[...] = sel[:, :K]


def _knn(feat, featT, bcol, brow, clo, chi):
    d = feat.shape[1]
    return pl.pallas_call(
        _knn_body,
        grid_spec=pltpu.PrefetchScalarGridSpec(
            num_scalar_prefetch=2,
            grid=(N // R,),
            in_specs=[
                pl.BlockSpec((R, d), lambda i, *_: (i, 0)),
                pl.BlockSpec((d, N), lambda i, *_: (0, 0)),
                pl.BlockSpec((R, 1), lambda i, *_: (i, 0)),
                pl.BlockSpec((1, N), lambda i, *_: (0, 0)),
            ],
            out_specs=pl.BlockSpec((R, K), lambda i, *_: (i, 0)),
            scratch_shapes=[pltpu.VMEM((R, N), jnp.float32)],
        ),
        out_shape=jax.ShapeDtypeStruct((N, K), jnp.int32),
    )(clo, chi, feat, featT, bcol, brow)


def _sc_gather(table, idx_flat):
    M = idx_flat.shape[0]
    D = table.shape[1]
    NW = 32
    per_w = M // NW
    CH = 512
    nch = per_w // CH
    mesh = plsc.VectorSubcoreMesh(core_axis_name="c", subcore_axis_name="s")

    @functools.partial(
        pl.kernel, mesh=mesh,
        out_type=jax.ShapeDtypeStruct((M, D), jnp.float32),
        scratch_types=[
            pltpu.VMEM((CH,), jnp.int32),
            pltpu.VMEM((CH, D), jnp.float32),
            pltpu.SemaphoreType.DMA,
        ],
    )
    def gk(table_hbm, idx_hbm, out_hbm, idx_v, rows_v, sem):
        wid = lax.axis_index("s") * 2 + lax.axis_index("c")
        base = wid * per_w

        for ci in range(nch):
            off = base + ci * CH
            pltpu.sync_copy(idx_hbm.at[pl.ds(off, CH)], idx_v)
            pltpu.async_copy(table_hbm.at[idx_v], rows_v, sem).wait()
            pltpu.sync_copy(rows_v, out_hbm.at[pl.ds(off, CH)])

    return gk(table, idx_flat)


def _pre1_body(pos_ref, x_ref, w1d_ref, w1b_ref, b1_ref,
               x0_ref, c1_ref, u1_ref):
    x0 = jnp.concatenate([pos_ref[...], 2.0 * x_ref[...] - 1.0], axis=1)
    x0_ref[...] = x0
    c1_ref[...] = lax.dot_general(x0, w1d_ref[...], (((1,), (0,)), ((), ())),
                                  preferred_element_type=jnp.float32) + b1_ref[...]
    u1 = lax.dot_general(x0, w1b_ref[...], (((1,), (0,)), ((), ())),
                         preferred_element_type=jnp.float32)
    u1_ref[...] = jnp.concatenate([u1, jnp.zeros((R, 64), jnp.float32)], axis=1)


def _pre1(pos, x, w1d, w1b, b1):
    return pl.pallas_call(
        _pre1_body,
        grid=(N // R,),
        in_specs=[
            pl.BlockSpec((R, 3), lambda i: (i, 0)),
            pl.BlockSpec((R, 1), lambda i: (i, 0)),
            pl.BlockSpec((4, 64), lambda i: (0, 0)),
            pl.BlockSpec((4, 64), lambda i: (0, 0)),
            pl.BlockSpec((1, 64), lambda i: (0, 0)),
        ],
        out_specs=[
            pl.BlockSpec((R, 4), lambda i: (i, 0)),
            pl.BlockSpec((R, 64), lambda i: (i, 0)),
            pl.BlockSpec((R, 128), lambda i: (i, 0)),
        ],
        out_shape=[
            jax.ShapeDtypeStruct((N, 4), jnp.float32),
            jax.ShapeDtypeStruct((N, 64), jnp.float32),
            jax.ShapeDtypeStruct((N, 128), jnp.float32),
        ],
    )(pos, x, w1d, w1b, b1)


def _mm(a, w_ref, b_ref=None):
    o = lax.dot_general(a, w_ref[...], (((1,), (0,)), ((), ())),
                        preferred_element_type=jnp.float32)
    if b_ref is not None:
        o = o + b_ref[...]
    return o


def _edge1_body(c1_ref, g_ref, w2_ref, b2_ref, w3_ref, b3_ref,
                w4d_ref, b4_ref, w4b_ref, x1_ref, c2_ref, u2_ref):
    c = c1_ref[...]
    acc = None
    for j in range(K):
        h = jnp.maximum(c + g_ref[:, j, :][:, :64], 0.0)
        h = jnp.maximum(_mm(h, w2_ref, b2_ref), 0.0)
        o = _mm(h, w3_ref, b3_ref)
        acc = o if acc is None else jnp.maximum(acc, o)
    x1_ref[...] = acc
    c2_ref[...] = _mm(acc, w4d_ref, b4_ref)
    u2_ref[...] = jnp.concatenate(
        [_mm(acc, w4b_ref), jnp.zeros((R, 64), jnp.float32)], axis=1)


def _edge1(c1, g, W2, b2, W3, b3, w4d, b4, w4b):
    return pl.pallas_call(
        _edge1_body,
        grid=(N // R,),
        in_specs=[
            pl.BlockSpec((R, 64), lambda i: (i, 0)),
            pl.BlockSpec((R, K, 128), lambda i: (i, 0, 0)),
            pl.BlockSpec((64, 64), lambda i: (0, 0)),
            pl.BlockSpec((1, 64), lambda i: (0, 0)),
            pl.BlockSpec((64, 64), lambda i: (0, 0)),
            pl.BlockSpec((1, 64), lambda i: (0, 0)),
            pl.BlockSpec((64, 64), lambda i: (0, 0)),
            pl.BlockSpec((1, 64), lambda i: (0, 0)),
            pl.BlockSpec((64, 64), lambda i: (0, 0)),
        ],
        out_specs=[
            pl.BlockSpec((R, 64), lambda i: (i, 0)),
            pl.BlockSpec((R, 64), lambda i: (i, 0)),
            pl.BlockSpec((R, 128), lambda i: (i, 0)),
        ],
        out_shape=[
            jax.ShapeDtypeStruct((N, 64), jnp.float32),
            jax.ShapeDtypeStruct((N, 64), jnp.float32),
            jax.ShapeDtypeStruct((N, 128), jnp.float32),
        ],
    )(c1, g, W2, b2, W3, b3, w4d, b4, w4b)


def _edgepre2_body(c2_ref, g_ref, w5d_ref, b5_ref, w5b_ref,
                   x2_ref, c3_ref, u3_ref):
    mx = g_ref[:, 0, :][:, :64]
    for j in range(1, K):
        mx = jnp.maximum(mx, g_ref[:, j, :][:, :64])
    x2 = c2_ref[...] + mx
    x2_ref[...] = x2
    c3_ref[...] = _mm(x2, w5d_ref, b5_ref)
    u3_ref[...] = _mm(x2, w5b_ref)


def _edgepre2(c2, g, w5d, b5, w5b):
    return pl.pallas_call(
        _edgepre2_body,
        grid=(N // R,),
        in_specs=[
            pl.BlockSpec((R, 64), lambda i: (i, 0)),
            pl.BlockSpec((R, K, 128), lambda i: (i, 0, 0)),
            pl.BlockSpec((64, 128), lambda i: (0, 0)),
            pl.BlockSpec((1, 128), lambda i: (0, 0)),
            pl.BlockSpec((64, 128), lambda i: (0, 0)),
        ],
        out_specs=[
            pl.BlockSpec((R, 64), lambda i: (i, 0)),
            pl.BlockSpec((R, 128), lambda i: (i, 0)),
            pl.BlockSpec((R, 128), lambda i: (i, 0)),
        ],
        out_shape=[
            jax.ShapeDtypeStruct((N, 64), jnp.float32),
            jax.ShapeDtypeStruct((N, 128), jnp.float32),
            jax.ShapeDtypeStruct((N, 128), jnp.float32),
        ],
    )(c2, g, w5d, b5, w5b)


def _final_body(c3_ref, g_ref, x1_ref, x2_ref, w6_ref, b6_ref, bcol_ref,
                pooled_ref):
    mx = g_ref[:, 0, :]
    for j in range(1, K):
        mx = jnp.maximum(mx, g_ref[:, j, :])
    x3 = c3_ref[...] + mx
    xcat = jnp.concatenate([x1_ref[...], x2_ref[...], x3], axis=1)
    o = _mm(xcat, w6_ref, b6_ref)

    @pl.when(pl.program_id(0) == 0)
    def _():
        pooled_ref[...] = jnp.full((NUM_GRAPHS, 1024), -jnp.inf, jnp.float32)

    bcol = bcol_ref[...]
    for gph in range(NUM_GRAPHS):
        mg = jnp.max(jnp.where(bcol == float(gph), o, -jnp.inf),
                     axis=0, keepdims=True)
        pooled_ref[gph:gph + 1, :] = jnp.maximum(pooled_ref[gph:gph + 1, :], mg)


def _final(c3, g, x1, x2, W6, b6, bcol):
    return pl.pallas_call(
        _final_body,
        grid=(N // R,),
        in_specs=[
            pl.BlockSpec((R, 128), lambda i: (i, 0)),
            pl.BlockSpec((R, K, 128), lambda i: (i, 0, 0)),
            pl.BlockSpec((R, 64), lambda i: (i, 0)),
            pl.BlockSpec((R, 64), lambda i: (i, 0)),
            pl.BlockSpec((256, 1024), lambda i: (0, 0)),
            pl.BlockSpec((1, 1024), lambda i: (0, 0)),
            pl.BlockSpec((R, 1), lambda i: (i, 0)),
        ],
        out_specs=pl.BlockSpec((NUM_GRAPHS, 1024), lambda i: (0, 0)),
        out_shape=jax.ShapeDtypeStruct((NUM_GRAPHS, 1024), jnp.float32),
    )(c3, g, x1, x2, W6, b6, bcol)


def _head_body(p_ref, w7_ref, b7_ref, w8_ref, b8_ref, w9_ref, b9_ref, out_ref):
    p = p_ref[...]
    p = jnp.where(p > -jnp.inf, p, 0.0)
    h = jnp.maximum(_mm(p, w7_ref, b7_ref), 0.0)
    h = jnp.maximum(_mm(h, w8_ref, b8_ref), 0.0)
    lg = _mm(h, w9_ref, b9_ref)
    m = jnp.max(lg, axis=1, keepdims=True)
    s = lg - m
    out_ref[...] = s - jnp.log(jnp.sum(jnp.exp(s), axis=1, keepdims=True))


def _head(pooled, W7, b7, W8, b8, W9, b9):
    return pl.pallas_call(
        _head_body,
        in_specs=[
            pl.BlockSpec((NUM_GRAPHS, 1024), lambda: (0, 0)),
            pl.BlockSpec((1024, 512), lambda: (0, 0)),
            pl.BlockSpec((1, 512), lambda: (0, 0)),
            pl.BlockSpec((512, 256), lambda: (0, 0)),
            pl.BlockSpec((1, 256), lambda: (0, 0)),
            pl.BlockSpec((256, 40), lambda: (0, 0)),
            pl.BlockSpec((1, 40), lambda: (0, 0)),
        ],
        out_specs=pl.BlockSpec((NUM_GRAPHS, 40), lambda: (0, 0)),
        out_shape=jax.ShapeDtypeStruct((NUM_GRAPHS, 40), jnp.float32),
    )(pooled, W7, b7, W8, b8, W9, b9)


def kernel(pos, x, batch, W1, b1, W2, b2, W3, b3, W4, b4, W5, b5, W6, b6,
           W7, b7, W8, b8, W9, b9):
    bf = batch.astype(jnp.float32)
    bcol = bf.reshape(N, 1)
    brow = bf.reshape(1, N)

    r0s = jnp.arange(N // R) * R
    blo = batch[r0s]
    bhi = batch[r0s + R - 1]
    col_lo = jnp.searchsorted(batch, blo, side="left").astype(jnp.int32)
    col_hi = jnp.searchsorted(batch, bhi, side="right").astype(jnp.int32)
    clo = col_lo // CW
    chi = (col_hi + CW - 1) // CW

    w1b = W1[4:]
    w1d = W1[:4] - w1b
    w4b = W4[64:]
    w4d = W4[:64] - w4b
    w5b = W5[64:]
    w5d = W5[:64] - w5b
    b1r, b2r, b3r, b4r, b5r, b6r, b7r, b8r, b9r = (
        b.reshape(1, -1) for b in (b1, b2, b3, b4, b5, b6, b7, b8, b9))

    x0, c1, u1 = _pre1(pos, x, w1d, w1b, b1r)
    idx1 = _knn(x0, x0.T, bcol, brow, clo, chi)
    g1 = _sc_gather(u1, idx1.reshape(-1)).reshape(N, K, 128)
    x1, c2, u2 = _edge1(c1, g1, W2, b2r, W3, b3r, w4d, b4r, w4b)

    idx2 = _knn(x1, x1.T, bcol, brow, clo, chi)
    g2 = _sc_gather(u2, idx2.reshape(-1)).reshape(N, K, 128)
    x2, c3, u3 = _edgepre2(c2, g2, w5d, b5r, w5b)

    idx3 = _knn(x2, x2.T, bcol, brow, clo, chi)
    g3 = _sc_gather(u3, idx3.reshape(-1)).reshape(N, K, 128)

    pooled = _final(c3, g3, x1, x2, W6, b6r, bcol)
    return _head(pooled, W7, b7r, W8, b8r, W9, b9r)


# speedup vs baseline: 11.3947x; 1.0762x over previous
import functools

import jax
import jax.numpy as jnp
from jax import lax
from jax.experimental import pallas as pl
from jax.experimental.pallas import tpu as pltpu
from jax.experimental.pallas import tpu_sc as plsc

N = 8192
K = 20
NUM_GRAPHS = 8
R = 512
HUGE = 3.0e38


CW = 512
IMAX = 2**31 - 1


def _knn_body(clo_ref, chi_ref, feat_ref, featT_ref, bcol_ref, brow_ref,
              idx_ref, dist_ref):
    pid = pl.program_id(0)
    c0 = clo_ref[pid]
    c1 = chi_ref[pid]
    f_r = feat_ref[...]
    sq_r = jnp.sum(f_r * f_r, axis=1, keepdims=True)
    bcol = bcol_ref[...]

    def compute_chunk(c, carry):
        sl = pl.ds(c * CW, CW)
        f_t = featT_ref[:, sl]
        d = lax.dot_general(f_r, f_t, (((1,), (0,)), ((), ())),
                            preferred_element_type=jnp.float32)
        sq_c = jnp.sum(f_t * f_t, axis=0, keepdims=True)
        dist = sq_r - 2.0 * d + sq_c
        dist = jnp.where(bcol != brow_ref[:, sl], 1e10, dist)
        dist_ref[:, sl] = dist
        return carry

    lax.fori_loop(c0, c1, compute_chunk, 0)

    kio = lax.broadcasted_iota(jnp.int32, (R, 32), 1)

    def select_one(t, carry):
        sel, aprev = carry

        def scan_chunk(c, inner):
            m, amin = inner
            sl = pl.ds(c * CW, CW)
            colio = lax.broadcasted_iota(jnp.int32, (R, CW), 1) + c * CW
            dch = jnp.where(colio == aprev, HUGE, dist_ref[:, sl])
            dist_ref[:, sl] = dch
            mch = jnp.min(dch, axis=1, keepdims=True)
            ach = jnp.min(jnp.where(dch == mch, colio, IMAX),
                          axis=1, keepdims=True)
            amin = jnp.where(mch < m, ach, amin)
            return jnp.minimum(m, mch), amin

        m, amin = lax.fori_loop(
            c0, c1, scan_chunk,
            (jnp.full((R, 1), HUGE, jnp.float32), jnp.zeros((R, 1), jnp.int32)))
        return jnp.where(kio == t, amin, sel), amin

    sel, _ = lax.fori_loop(
        0, K, select_one,
        (jnp.zeros((R, 32), jnp.int32), jnp.full((R, 1), -1, jnp.int32)))
    idx_ref[...] = sel[:, :K]


def _knn(feat, featT, bcol, brow, clo, chi):
    d = feat.shape[1]
    return pl.pallas_call(
        _knn_body,
        grid_spec=pltpu.PrefetchScalarGridSpec(
            num_scalar_prefetch=2,
            grid=(N // R,),
            in_specs=[
                pl.BlockSpec((R, d), lambda i, *_: (i, 0)),
                pl.BlockSpec((d, N), lambda i, *_: (0, 0)),
                pl.BlockSpec((R, 1), lambda i, *_: (i, 0)),
                pl.BlockSpec((1, N), lambda i, *_: (0, 0)),
            ],
            out_specs=pl.BlockSpec((R, K), lambda i, *_: (i, 0)),
            scratch_shapes=[pltpu.VMEM((R, N), jnp.float32)],
        ),
        out_shape=jax.ShapeDtypeStruct((N, K), jnp.int32),
    )(clo, chi, feat, featT, bcol, brow)


def _sc_gather(table, idx_flat):
    M = idx_flat.shape[0]
    D = table.shape[1]
    NW = 32
    per_w = M // NW
    CH = 512
    nch = per_w // CH
    mesh = plsc.VectorSubcoreMesh(core_axis_name="c", subcore_axis_name="s")

    @functools.partial(
        pl.kernel, mesh=mesh,
        out_type=jax.ShapeDtypeStruct((M, D), jnp.float32),
        scratch_types=[
            pltpu.VMEM((CH,), jnp.int32),
            pltpu.VMEM((CH, D), jnp.float32),
            pltpu.SemaphoreType.DMA,
        ],
    )
    def gk(table_hbm, idx_hbm, out_hbm, idx_v, rows_v, sem):
        wid = lax.axis_index("s") * 2 + lax.axis_index("c")
        base = wid * per_w

        for ci in range(nch):
            off = base + ci * CH
            pltpu.sync_copy(idx_hbm.at[pl.ds(off, CH)], idx_v)
            pltpu.async_copy(table_hbm.at[idx_v], rows_v, sem).wait()
            pltpu.sync_copy(rows_v, out_hbm.at[pl.ds(off, CH)])

    return gk(table, idx_flat)


def _pre1_body(pos_ref, x_ref, w1d_ref, w1b_ref, b1_ref,
               x0_ref, c1_ref, u1_ref):
    x0 = jnp.concatenate([pos_ref[...], 2.0 * x_ref[...] - 1.0], axis=1)
    x0_ref[...] = x0
    c1_ref[...] = lax.dot_general(x0, w1d_ref[...], (((1,), (0,)), ((), ())),
                                  preferred_element_type=jnp.float32) + b1_ref[...]
    u1 = lax.dot_general(x0, w1b_ref[...], (((1,), (0,)), ((), ())),
                         preferred_element_type=jnp.float32)
    u1_ref[...] = jnp.concatenate([u1, jnp.zeros((R, 64), jnp.float32)], axis=1)


def _pre1(pos, x, w1d, w1b, b1):
    return pl.pallas_call(
        _pre1_body,
        grid=(N // R,),
        in_specs=[
            pl.BlockSpec((R, 3), lambda i: (i, 0)),
            pl.BlockSpec((R, 1), lambda i: (i, 0)),
            pl.BlockSpec((4, 64), lambda i: (0, 0)),
            pl.BlockSpec((4, 64), lambda i: (0, 0)),
            pl.BlockSpec((1, 64), lambda i: (0, 0)),
        ],
        out_specs=[
            pl.BlockSpec((R, 4), lambda i: (i, 0)),
            pl.BlockSpec((R, 64), lambda i: (i, 0)),
            pl.BlockSpec((R, 128), lambda i: (i, 0)),
        ],
        out_shape=[
            jax.ShapeDtypeStruct((N, 4), jnp.float32),
            jax.ShapeDtypeStruct((N, 64), jnp.float32),
            jax.ShapeDtypeStruct((N, 128), jnp.float32),
        ],
    )(pos, x, w1d, w1b, b1)


def _mm(a, w_ref, b_ref=None):
    o = lax.dot_general(a, w_ref[...], (((1,), (0,)), ((), ())),
                        preferred_element_type=jnp.float32)
    if b_ref is not None:
        o = o + b_ref[...]
    return o


def _edge1_body(c1_ref, g_ref, w2_ref, b2_ref, w3_ref, b3_ref,
                w4d_ref, b4_ref, w4b_ref, x1_ref, c2_ref, u2_ref):
    c = c1_ref[...]
    acc = None
    for j in range(K):
        h = jnp.maximum(c + g_ref[:, j, :][:, :64], 0.0)
        h = jnp.maximum(_mm(h, w2_ref, b2_ref), 0.0)
        o = _mm(h, w3_ref, b3_ref)
        acc = o if acc is None else jnp.maximum(acc, o)
    x1_ref[...] = acc
    c2_ref[...] = _mm(acc, w4d_ref, b4_ref)
    u2_ref[...] = jnp.concatenate(
        [_mm(acc, w4b_ref), jnp.zeros((R, 64), jnp.float32)], axis=1)


def _edge1(c1, g, W2, b2, W3, b3, w4d, b4, w4b):
    return pl.pallas_call(
        _edge1_body,
        grid=(N // R,),
        in_specs=[
            pl.BlockSpec((R, 64), lambda i: (i, 0)),
            pl.BlockSpec((R, K, 128), lambda i: (i, 0, 0)),
            pl.BlockSpec((64, 64), lambda i: (0, 0)),
            pl.BlockSpec((1, 64), lambda i: (0, 0)),
            pl.BlockSpec((64, 64), lambda i: (0, 0)),
            pl.BlockSpec((1, 64), lambda i: (0, 0)),
            pl.BlockSpec((64, 64), lambda i: (0, 0)),
            pl.BlockSpec((1, 64), lambda i: (0, 0)),
            pl.BlockSpec((64, 64), lambda i: (0, 0)),
        ],
        out_specs=[
            pl.BlockSpec((R, 64), lambda i: (i, 0)),
            pl.BlockSpec((R, 64), lambda i: (i, 0)),
            pl.BlockSpec((R, 128), lambda i: (i, 0)),
        ],
        out_shape=[
            jax.ShapeDtypeStruct((N, 64), jnp.float32),
            jax.ShapeDtypeStruct((N, 64), jnp.float32),
            jax.ShapeDtypeStruct((N, 128), jnp.float32),
        ],
    )(c1, g, W2, b2, W3, b3, w4d, b4, w4b)


def _edgepre2_body(c2_ref, g_ref, w5d_ref, b5_ref, w5b_ref,
                   x2_ref, c3_ref, u3_ref):
    mx = g_ref[:, 0, :][:, :64]
    for j in range(1, K):
        mx = jnp.maximum(mx, g_ref[:, j, :][:, :64])
    x2 = c2_ref[...] + mx
    x2_ref[...] = x2
    c3_ref[...] = _mm(x2, w5d_ref, b5_ref)
    u3_ref[...] = _mm(x2, w5b_ref)


def _edgepre2(c2, g, w5d, b5, w5b):
    return pl.pallas_call(
        _edgepre2_body,
        grid=(N // R,),
        in_specs=[
            pl.BlockSpec((R, 64), lambda i: (i, 0)),
            pl.BlockSpec((R, K, 128), lambda i: (i, 0, 0)),
            pl.BlockSpec((64, 128), lambda i: (0, 0)),
            pl.BlockSpec((1, 128), lambda i: (0, 0)),
            pl.BlockSpec((64, 128), lambda i: (0, 0)),
        ],
        out_specs=[
            pl.BlockSpec((R, 64), lambda i: (i, 0)),
            pl.BlockSpec((R, 128), lambda i: (i, 0)),
            pl.BlockSpec((R, 128), lambda i: (i, 0)),
        ],
        out_shape=[
            jax.ShapeDtypeStruct((N, 64), jnp.float32),
            jax.ShapeDtypeStruct((N, 128), jnp.float32),
            jax.ShapeDtypeStruct((N, 128), jnp.float32),
        ],
    )(c2, g, w5d, b5, w5b)


def _final_body(c3_ref, g_ref, x1_ref, x2_ref, w6_ref, b6_ref, bcol_ref,
                pooled_ref):
    mx = g_ref[:, 0, :]
    for j in range(1, K):
        mx = jnp.maximum(mx, g_ref[:, j, :])
    x3 = c3_ref[...] + mx
    xcat = jnp.concatenate([x1_ref[...], x2_ref[...], x3], axis=1)
    o = _mm(xcat, w6_ref, b6_ref)

    @pl.when(pl.program_id(0) == 0)
    def _():
        pooled_ref[...] = jnp.full((NUM_GRAPHS, 1024), -jnp.inf, jnp.float32)

    bcol = bcol_ref[...]
    for gph in range(NUM_GRAPHS):
        mg = jnp.max(jnp.where(bcol == float(gph), o, -jnp.inf),
                     axis=0, keepdims=True)
        pooled_ref[gph:gph + 1, :] = jnp.maximum(pooled_ref[gph:gph + 1, :], mg)


def _final(c3, g, x1, x2, W6, b6, bcol):
    return pl.pallas_call(
        _final_body,
        grid=(N // R,),
        in_specs=[
            pl.BlockSpec((R, 128), lambda i: (i, 0)),
            pl.BlockSpec((R, K, 128), lambda i: (i, 0, 0)),
            pl.BlockSpec((R, 64), lambda i: (i, 0)),
            pl.BlockSpec((R, 64), lambda i: (i, 0)),
            pl.BlockSpec((256, 1024), lambda i: (0, 0)),
            pl.BlockSpec((1, 1024), lambda i: (0, 0)),
            pl.BlockSpec((R, 1), lambda i: (i, 0)),
        ],
        out_specs=pl.BlockSpec((NUM_GRAPHS, 1024), lambda i: (0, 0)),
        out_shape=jax.ShapeDtypeStruct((NUM_GRAPHS, 1024), jnp.float32),
    )(c3, g, x1, x2, W6, b6, bcol)


def _head_body(p_ref, w7_ref, b7_ref, w8_ref, b8_ref, w9_ref, b9_ref, out_ref):
    p = p_ref[...]
    p = jnp.where(p > -jnp.inf, p, 0.0)
    h = jnp.maximum(_mm(p, w7_ref, b7_ref), 0.0)
    h = jnp.maximum(_mm(h, w8_ref, b8_ref), 0.0)
    lg = _mm(h, w9_ref, b9_ref)
    m = jnp.max(lg, axis=1, keepdims=True)
    s = lg - m
    out_ref[...] = s - jnp.log(jnp.sum(jnp.exp(s), axis=1, keepdims=True))


def _head(pooled, W7, b7, W8, b8, W9, b9):
    return pl.pallas_call(
        _head_body,
        in_specs=[
            pl.BlockSpec((NUM_GRAPHS, 1024), lambda: (0, 0)),
            pl.BlockSpec((1024, 512), lambda: (0, 0)),
            pl.BlockSpec((1, 512), lambda: (0, 0)),
            pl.BlockSpec((512, 256), lambda: (0, 0)),
            pl.BlockSpec((1, 256), lambda: (0, 0)),
            pl.BlockSpec((256, 40), lambda: (0, 0)),
            pl.BlockSpec((1, 40), lambda: (0, 0)),
        ],
        out_specs=pl.BlockSpec((NUM_GRAPHS, 40), lambda: (0, 0)),
        out_shape=jax.ShapeDtypeStruct((NUM_GRAPHS, 40), jnp.float32),
    )(pooled, W7, b7, W8, b8, W9, b9)


def kernel(pos, x, batch, W1, b1, W2, b2, W3, b3, W4, b4, W5, b5, W6, b6,
           W7, b7, W8, b8, W9, b9):
    bf = batch.astype(jnp.float32)
    bcol = bf.reshape(N, 1)
    brow = bf.reshape(1, N)

    r0s = jnp.arange(N // R) * R
    blo = batch[r0s]
    bhi = batch[r0s + R - 1]
    col_lo = jnp.searchsorted(batch, blo, side="left").astype(jnp.int32)
    col_hi = jnp.searchsorted(batch, bhi, side="right").astype(jnp.int32)
    clo = col_lo // CW
    chi = (col_hi + CW - 1) // CW

    w1b = W1[4:]
    w1d = W1[:4] - w1b
    w4b = W4[64:]
    w4d = W4[:64] - w4b
    w5b = W5[64:]
    w5d = W5[:64] - w5b
    b1r, b2r, b3r, b4r, b5r, b6r, b7r, b8r, b9r = (
        b.reshape(1, -1) for b in (b1, b2, b3, b4, b5, b6, b7, b8, b9))

    x0, c1, u1 = _pre1(pos, x, w1d, w1b, b1r)
    idx1 = _knn(x0, x0.T, bcol, brow, clo, chi)
    g1 = _sc_gather(u1, idx1.reshape(-1)).reshape(N, K, 128)
    x1, c2, u2 = _edge1(c1, g1, W2, b2r, W3, b3r, w4d, b4r, w4b)

    idx2 = _knn(x1, x1.T, bcol, brow, clo, chi)
    g2 = _sc_gather(u2, idx2.reshape(-1)).reshape(N, K, 128)
    x2, c3, u3 = _edgepre2(c2, g2, w5d, b5r, w5b)

    idx3 = _knn(x2, x2.T, bcol, brow, clo, chi)
    g3 = _sc_gather(u3, idx3.reshape(-1)).reshape(N, K, 128)

    pooled = _final(c3, g3, x1, x2, W6, b6r, bcol)
    return _head(pooled, W7, b7r, W8, b8r, W9, b9r)
